# baseline (speedup 1.0000x reference)
import functools

import jax
import jax.numpy as jnp
from jax import lax
from jax.experimental import pallas as pl
from jax.experimental.pallas import tpu as pltpu
from jax.experimental.pallas import tpu_sc as plsc

_TM = 1024


_NC = 4


def _dist_argmin_body(x2_ref, e2_ref, x_ref, emb_ref, idx_ref):
    x = x_ref[0]
    x2 = x2_ref[0]
    k = emb_ref.shape[0]
    kc = k // _NC
    best_m = None
    best_i = None
    for c in range(_NC):
        emb = emb_ref[pl.ds(c * kc, kc), :]
        xe = lax.dot_general(
            x, emb, (((1,), (1,)), ((), ())),
            preferred_element_type=jnp.float32)
        d = (x2 + e2_ref[:, pl.ds(c * kc, kc)]) - 2.0 * xe
        m = jnp.min(d, axis=1, keepdims=True)
        iot = lax.broadcasted_iota(jnp.int32, d.shape, 1).astype(jnp.float32)
        cand = jnp.where(d == m, iot, jnp.float32(kc))
        idxf = jnp.min(cand, axis=1, keepdims=True) + jnp.float32(c * kc)
        if c == 0:
            best_m, best_i = m, idxf
        else:
            take = m < best_m
            best_m = jnp.minimum(best_m, m)
            best_i = jnp.where(take, idxf, best_i)
    idx_ref[0] = best_i.astype(jnp.int32)


def _nearest_code_indices(x2, e2, x, embeddings):
    bsz, t, d = x.shape
    k = embeddings.shape[0]
    grid = (bsz * t // _TM,)
    tb = _TM // t if _TM > t else 1
    return pl.pallas_call(
        _dist_argmin_body,
        grid=grid,
        in_specs=[
            pl.BlockSpec((tb, _TM // tb, 1), lambda i: (i, 0, 0)),
            pl.BlockSpec((1, k), lambda i: (0, 0)),
            pl.BlockSpec((tb, _TM // tb, d), lambda i: (i, 0, 0)),
            pl.BlockSpec((k, d), lambda i: (0, 0)),
        ],
        out_specs=pl.BlockSpec((tb, _TM // tb, 1), lambda i: (i, 0, 0)),
        out_shape=jax.ShapeDtypeStruct((bsz, t, 1), jnp.int32),
        compiler_params=pltpu.CompilerParams(
            dimension_semantics=("parallel",)),
    )(x2, e2, x, embeddings)


def _gather_rows(table, idx, bsz, t):
    b = idx.shape[0]
    d = table.shape[1]
    info = plsc.get_sparse_core_info()
    nw = info.num_cores * info.num_subcores
    b_per_w = b // nw
    t_per_w = t // b_per_w
    mesh = plsc.VectorSubcoreMesh(core_axis_name="c", subcore_axis_name="s")
    out_sd = jax.ShapeDtypeStruct((bsz, t, d), jnp.float32)

    @functools.partial(
        pl.kernel, mesh=mesh,
        out_type=(out_sd, out_sd),
        scratch_types=[
            pltpu.VMEM((b_per_w,), jnp.int32),
            pltpu.VMEM((b_per_w, d), jnp.float32),
            pltpu.SemaphoreType.DMA,
        ],
    )
    def gather_kernel(table_hbm, idx_hbm, out0_hbm, out1_hbm,
                      idx_v, rows_v, sem):
        wid = lax.axis_index("s") * info.num_cores + lax.axis_index("c")
        base = wid * b_per_w
        row = wid // t_per_w
        col = (wid % t_per_w) * b_per_w
        pltpu.sync_copy(idx_hbm.at[pl.ds(base, b_per_w)], idx_v)
        pltpu.async_copy(table_hbm.at[idx_v], rows_v, sem).wait()
        w0 = pltpu.async_copy(rows_v, out0_hbm.at[row, pl.ds(col, b_per_w)], sem)
        w1 = pltpu.async_copy(rows_v, out1_hbm.at[row, pl.ds(col, b_per_w)], sem)
        w0.wait()
        w1.wait()

    return gather_kernel(table, idx)


def kernel(x, embeddings):
    bsz, t, d = x.shape
    m = bsz * t
    x2 = jnp.sum(x * x, axis=-1, keepdims=True)
    e2 = jnp.sum(embeddings * embeddings, axis=-1)[None]
    ind = _nearest_code_indices(x2, e2, x, embeddings)
    out0, out1 = _gather_rows(embeddings, ind.reshape(m), bsz, t)
    return (out0, out1)
